# baseline (speedup 1.0000x reference)
import functools

import jax
import jax.numpy as jnp
from jax import lax
from jax.experimental import pallas as pl
from jax.experimental.pallas import tpu as pltpu
from jax.experimental.pallas import tpu_sc as plsc

N = 10000
E = 320000
D = 128

NC = 2
NS = 16
NW = NC * NS
CHUNK = 128
ECHUNKS = E // CHUNK
NCHUNK = 78
NEXTRA = ECHUNKS - NCHUNK * NW
NWIN = 3
WCH = NCHUNK // NWIN
WSZ = WCH * CHUNK
ROWS_T = 624
ROWS_TAIL = N - ROWS_T * NS
ZB = 800


def _make_sc_agg(compute_deg: bool):
    out_type = [jax.ShapeDtypeStruct((NC, N, D), jnp.float32)]
    if compute_deg:
        out_type.append(jax.ShapeDtypeStruct((NC, N), jnp.float32))

    scratch = [
        pltpu.VMEM((2, WSZ), jnp.int32),
        pltpu.VMEM((2, WSZ), jnp.int32),
        pltpu.VMEM((1, CHUNK), jnp.int32),
        pltpu.VMEM((1, CHUNK), jnp.int32),
        pltpu.VMEM((CHUNK, D), jnp.float32),
        pltpu.VMEM((CHUNK, D), jnp.float32),
        pltpu.VMEM((CHUNK,), jnp.float32),
        pltpu.VMEM((ZB,), jnp.float32),
        pltpu.VMEM_SHARED((N, D), jnp.float32),
        pltpu.VMEM_SHARED((N,), jnp.float32),
        pltpu.SemaphoreType.DMA,
        pltpu.SemaphoreType.DMA,
        pltpu.SemaphoreType.DMA,
    ]

    mesh = plsc.VectorSubcoreMesh(core_axis_name="c", subcore_axis_name="s")

    def body(x_hbm, e_hbm, *rest):
        if compute_deg:
            agg_out, deg_out = rest[0], rest[1]
            scr = rest[2:]
        else:
            agg_out = rest[0]
            deg_out = None
            scr = rest[1:]
        (src_v, dst_v, srcx_v, dstx_v, rows0, rows1, ones_v, z1_v,
         agg_s, deg_s, sem0, sem1, semw) = scr

        c = lax.axis_index("c")
        s = lax.axis_index("s")
        wid = s * NC + c
        eofs = pl.multiple_of(
            jnp.where(wid < NEXTRA, wid * (NCHUNK + 1),
                      NEXTRA * (NCHUNK + 1)
                      + (wid - NEXTRA) * NCHUNK) * CHUNK,
            CHUNK)

        z16 = jnp.zeros((16,), jnp.float32)

        pltpu.sync_copy(e_hbm.at[0, pl.ds(eofs, WSZ)], src_v.at[0])
        pltpu.sync_copy(e_hbm.at[1, pl.ds(eofs, WSZ)], dst_v.at[0])
        pltpu.async_copy(x_hbm.at[src_v.at[0, pl.ds(0, CHUNK)]], rows0, sem0)

        def zrow(i, carry):
            for j in range(D // 16):
                rows1[i, pl.ds(j * 16, 16)] = z16
            return carry

        lax.fori_loop(0, CHUNK, zrow, 0)

        r0 = s * ROWS_T
        n_full = ROWS_T // CHUNK
        rem = ROWS_T - n_full * CHUNK
        for t in range(n_full):
            pltpu.sync_copy(rows1, agg_s.at[pl.ds(r0 + t * CHUNK, CHUNK)])
        if rem:
            pltpu.sync_copy(rows1.at[pl.ds(0, rem)],
                            agg_s.at[pl.ds(r0 + n_full * CHUNK, rem)])

        @pl.when(s == 0)
        def _():
            pltpu.sync_copy(rows1.at[pl.ds(0, ROWS_TAIL)],
                            agg_s.at[pl.ds(ROWS_T * NS, ROWS_TAIL)])

        if compute_deg:
            def zz(i, carry):
                z1_v[pl.ds(i * 16, 16)] = z16
                return carry
            lax.fori_loop(0, ZB // 16, zz, 0)

            one16 = jnp.ones((16,), jnp.float32)
            for j in range(CHUNK // 16):
                ones_v[pl.ds(j * 16, 16)] = one16

            @pl.when(s == 0)
            def _():
                nf = N // ZB
                for t in range(nf):
                    pltpu.sync_copy(z1_v, deg_s.at[pl.ds(t * ZB, ZB)])
                drem = N - nf * ZB
                if drem:
                    pltpu.sync_copy(z1_v.at[pl.ds(0, drem)],
                                    deg_s.at[pl.ds(nf * ZB, drem)])

        plsc.subcore_barrier()

        def idx(v, b, k):
            return v.at[b, pl.ds(k * CHUNK, CHUNK)]

        def gather(b, k, buf, sem):
            return pltpu.async_copy(x_hbm.at[idx(src_v, b, k)], buf, sem)

        def scat(b, k, buf):
            pltpu.sync_copy(buf, agg_s.at[idx(dst_v, b, k)], add=True)
            if compute_deg:
                pltpu.sync_copy(ones_v, deg_s.at[idx(dst_v, b, k)], add=True)

        def make_step(b):
            def step(g, carry):
                k0 = 2 * g
                gather(b, k0 + 1, rows1, sem1)
                pltpu.make_async_copy(x_hbm.at[idx(src_v, b, k0)], rows0,
                                      sem0).wait()
                scat(b, k0, rows0)

                @pl.when(k0 + 2 < WCH)
                def _():
                    gather(b, k0 + 2, rows0, sem0)

                pltpu.make_async_copy(x_hbm.at[idx(src_v, b, k0 + 1)], rows1,
                                      sem1).wait()
                scat(b, k0 + 1, rows1)
                return carry
            return step

        for w in range(NWIN):
            b = w % 2
            if w + 1 < NWIN:
                nb = (w + 1) % 2
                wofs = eofs + (w + 1) * WSZ
                pltpu.async_copy(e_hbm.at[0, pl.ds(wofs, WSZ)],
                                 src_v.at[nb], semw)
                pltpu.async_copy(e_hbm.at[1, pl.ds(wofs, WSZ)],
                                 dst_v.at[nb], semw)
            lax.fori_loop(0, WCH // 2, make_step(b), 0)
            if w + 1 < NWIN:
                nb = (w + 1) % 2
                wofs = eofs + (w + 1) * WSZ
                pltpu.make_async_copy(e_hbm.at[0, pl.ds(wofs, WSZ)],
                                      src_v.at[nb], semw).wait()
                pltpu.make_async_copy(e_hbm.at[1, pl.ds(wofs, WSZ)],
                                      dst_v.at[nb], semw).wait()
                gather(nb, 0, rows0, sem0)

        @pl.when(wid < NEXTRA)
        def _():
            xofs = eofs + NCHUNK * CHUNK
            pltpu.sync_copy(e_hbm.at[0, pl.ds(xofs, CHUNK)], srcx_v.at[0])
            pltpu.sync_copy(e_hbm.at[1, pl.ds(xofs, CHUNK)], dstx_v.at[0])
            pltpu.async_copy(x_hbm.at[srcx_v.at[0]], rows0, sem0).wait()
            pltpu.sync_copy(rows0, agg_s.at[dstx_v.at[0]], add=True)
            if compute_deg:
                pltpu.sync_copy(ones_v, deg_s.at[dstx_v.at[0]], add=True)

        plsc.subcore_barrier()

        pltpu.sync_copy(agg_s.at[pl.ds(r0, ROWS_T)],
                        agg_out.at[c, pl.ds(r0, ROWS_T)])

        @pl.when(s == 0)
        def _():
            pltpu.sync_copy(agg_s.at[pl.ds(ROWS_T * NS, ROWS_TAIL)],
                            agg_out.at[c, pl.ds(ROWS_T * NS, ROWS_TAIL)])

        if compute_deg:
            @pl.when(s == 0)
            def _():
                pltpu.sync_copy(deg_s, deg_out.at[c])

    return pl.kernel(body, out_type=out_type, scratch_types=scratch, mesh=mesh)


_sc_agg_deg = _make_sc_agg(True)
_sc_agg = _make_sc_agg(False)


RB = 2000
NB = N // RB


def _combine(aggp, degp, x, wl, bl, wr):
    agg = aggp[0] + aggp[1]
    deg = jnp.maximum(degp[:, 0:1] + degp[:, 1:2], 1.0)
    mean = agg / deg
    return (lax.dot_general(mean, wl[...], (((1,), (1,)), ((), ())),
                            preferred_element_type=jnp.float32)
            + lax.dot_general(x[...], wr[...], (((1,), (1,)), ((), ())),
                              preferred_element_type=jnp.float32)
            + bl[...])


def _tc_hidden_body(aggp, degp, x, wl, bl, wr, o):
    z = _combine(aggp, degp[...], x, wl, bl, wr)
    o[...] = jnp.maximum(z, 0.0)


def _tc_final_body(aggp, degp, x, wl, bl, wr, o):
    z = _combine(aggp, degp[...], x, wl, bl, wr)
    m = jnp.max(z, axis=-1, keepdims=True)
    lse = jnp.log(jnp.sum(jnp.exp(z - m), axis=-1, keepdims=True)) + m
    o[...] = z - lse


def _tc_layer(body, aggp, degp, x, wl, bl, wr):
    return pl.pallas_call(
        body,
        grid=(NB,),
        in_specs=[
            pl.BlockSpec((NC, RB, D), lambda i: (0, i, 0)),
            pl.BlockSpec((RB, NC), lambda i: (i, 0)),
            pl.BlockSpec((RB, D), lambda i: (i, 0)),
            pl.BlockSpec((D, D), lambda i: (0, 0)),
            pl.BlockSpec((1, D), lambda i: (0, 0)),
            pl.BlockSpec((D, D), lambda i: (0, 0)),
        ],
        out_specs=pl.BlockSpec((RB, D), lambda i: (i, 0)),
        out_shape=jax.ShapeDtypeStruct((N, D), jnp.float32),
    )(aggp, degp, x, wl, bl, wr)


def kernel(x, edge_index, Wl1, bl1, Wr1, Wl2, bl2, Wr2):
    ei = edge_index.astype(jnp.int32)

    aggp1, degp = _sc_agg_deg(x, ei)
    degpt = degp.T
    h = _tc_layer(_tc_hidden_body, aggp1, degpt, x,
                  Wl1, bl1.reshape(1, D), Wr1)
    aggp2, _unused_deg = _sc_agg_deg(h, ei)
    out = _tc_layer(_tc_final_body, aggp2, degpt, h,
                    Wl2, bl2.reshape(1, D), Wr2)
    return out


# speedup vs baseline: 1.0135x; 1.0135x over previous
import functools

import jax
import jax.numpy as jnp
from jax import lax
from jax.experimental import pallas as pl
from jax.experimental.pallas import tpu as pltpu
from jax.experimental.pallas import tpu_sc as plsc

N = 10000
E = 320000
D = 128

NC = 2
NS = 16
NW = NC * NS
CHUNK = 128
ECHUNKS = E // CHUNK
NCHUNK = 78
NEXTRA = ECHUNKS - NCHUNK * NW
NWIN = 3
WCH = NCHUNK // NWIN
WSZ = WCH * CHUNK
ROWS_T = 624
ROWS_TAIL = N - ROWS_T * NS
ZB = 800


def _make_sc_agg(compute_deg: bool):
    out_type = [jax.ShapeDtypeStruct((NC, N, D), jnp.float32)]
    if compute_deg:
        out_type.append(jax.ShapeDtypeStruct((NC, N), jnp.float32))

    scratch = [
        pltpu.VMEM((2, WSZ), jnp.int32),
        pltpu.VMEM((2, WSZ), jnp.int32),
        pltpu.VMEM((1, CHUNK), jnp.int32),
        pltpu.VMEM((1, CHUNK), jnp.int32),
        pltpu.VMEM((CHUNK, D), jnp.float32),
        pltpu.VMEM((CHUNK, D), jnp.float32),
        pltpu.VMEM((CHUNK,), jnp.float32),
        pltpu.VMEM((ZB,), jnp.float32),
        pltpu.VMEM_SHARED((N, D), jnp.float32),
        pltpu.VMEM_SHARED((N,), jnp.float32),
        pltpu.SemaphoreType.DMA,
        pltpu.SemaphoreType.DMA,
        pltpu.SemaphoreType.DMA,
    ]

    mesh = plsc.VectorSubcoreMesh(core_axis_name="c", subcore_axis_name="s")

    def body(x_hbm, e_hbm, *rest):
        if compute_deg:
            agg_out, deg_out = rest[0], rest[1]
            scr = rest[2:]
        else:
            agg_out = rest[0]
            deg_out = None
            scr = rest[1:]
        (src_v, dst_v, srcx_v, dstx_v, rows0, rows1, ones_v, z1_v,
         agg_s, deg_s, sem0, sem1, semw) = scr

        c = lax.axis_index("c")
        s = lax.axis_index("s")
        wid = s * NC + c
        eofs = pl.multiple_of(
            jnp.where(wid < NEXTRA, wid * (NCHUNK + 1),
                      NEXTRA * (NCHUNK + 1)
                      + (wid - NEXTRA) * NCHUNK) * CHUNK,
            CHUNK)

        z16 = jnp.zeros((16,), jnp.float32)

        pltpu.sync_copy(e_hbm.at[0, pl.ds(eofs, WSZ)], src_v.at[0])
        pltpu.sync_copy(e_hbm.at[1, pl.ds(eofs, WSZ)], dst_v.at[0])
        pltpu.async_copy(x_hbm.at[src_v.at[0, pl.ds(0, CHUNK)]], rows0, sem0)

        def zrow(i, carry):
            for j in range(D // 16):
                rows1[i, pl.ds(j * 16, 16)] = z16
            return carry

        lax.fori_loop(0, CHUNK, zrow, 0)

        r0 = s * ROWS_T
        n_full = ROWS_T // CHUNK
        rem = ROWS_T - n_full * CHUNK
        for t in range(n_full):
            pltpu.sync_copy(rows1, agg_s.at[pl.ds(r0 + t * CHUNK, CHUNK)])
        if rem:
            pltpu.sync_copy(rows1.at[pl.ds(0, rem)],
                            agg_s.at[pl.ds(r0 + n_full * CHUNK, rem)])

        @pl.when(s == 0)
        def _():
            pltpu.sync_copy(rows1.at[pl.ds(0, ROWS_TAIL)],
                            agg_s.at[pl.ds(ROWS_T * NS, ROWS_TAIL)])

        if compute_deg:
            def zz(i, carry):
                z1_v[pl.ds(i * 16, 16)] = z16
                return carry
            lax.fori_loop(0, ZB // 16, zz, 0)

            one16 = jnp.ones((16,), jnp.float32)
            for j in range(CHUNK // 16):
                ones_v[pl.ds(j * 16, 16)] = one16

            @pl.when(s == 0)
            def _():
                nf = N // ZB
                for t in range(nf):
                    pltpu.sync_copy(z1_v, deg_s.at[pl.ds(t * ZB, ZB)])
                drem = N - nf * ZB
                if drem:
                    pltpu.sync_copy(z1_v.at[pl.ds(0, drem)],
                                    deg_s.at[pl.ds(nf * ZB, drem)])

        plsc.subcore_barrier()

        def idx(v, b, k):
            return v.at[b, pl.ds(k * CHUNK, CHUNK)]

        def gather(b, k, buf, sem):
            return pltpu.async_copy(x_hbm.at[idx(src_v, b, k)], buf, sem)

        def scat(b, k, buf):
            pltpu.sync_copy(buf, agg_s.at[idx(dst_v, b, k)], add=True)
            if compute_deg:
                pltpu.sync_copy(ones_v, deg_s.at[idx(dst_v, b, k)], add=True)

        def make_step(b):
            def step(g, carry):
                k0 = 2 * g
                gather(b, k0 + 1, rows1, sem1)
                pltpu.make_async_copy(x_hbm.at[idx(src_v, b, k0)], rows0,
                                      sem0).wait()
                scat(b, k0, rows0)

                @pl.when(k0 + 2 < WCH)
                def _():
                    gather(b, k0 + 2, rows0, sem0)

                pltpu.make_async_copy(x_hbm.at[idx(src_v, b, k0 + 1)], rows1,
                                      sem1).wait()
                scat(b, k0 + 1, rows1)
                return carry
            return step

        for w in range(NWIN):
            b = w % 2
            if w + 1 < NWIN:
                nb = (w + 1) % 2
                wofs = eofs + (w + 1) * WSZ
                pltpu.async_copy(e_hbm.at[0, pl.ds(wofs, WSZ)],
                                 src_v.at[nb], semw)
                pltpu.async_copy(e_hbm.at[1, pl.ds(wofs, WSZ)],
                                 dst_v.at[nb], semw)
            lax.fori_loop(0, WCH // 2, make_step(b), 0)
            if w + 1 < NWIN:
                nb = (w + 1) % 2
                wofs = eofs + (w + 1) * WSZ
                pltpu.make_async_copy(e_hbm.at[0, pl.ds(wofs, WSZ)],
                                      src_v.at[nb], semw).wait()
                pltpu.make_async_copy(e_hbm.at[1, pl.ds(wofs, WSZ)],
                                      dst_v.at[nb], semw).wait()
                gather(nb, 0, rows0, sem0)

        @pl.when(wid < NEXTRA)
        def _():
            xofs = eofs + NCHUNK * CHUNK
            pltpu.sync_copy(e_hbm.at[0, pl.ds(xofs, CHUNK)], srcx_v.at[0])
            pltpu.sync_copy(e_hbm.at[1, pl.ds(xofs, CHUNK)], dstx_v.at[0])
            pltpu.async_copy(x_hbm.at[srcx_v.at[0]], rows0, sem0).wait()
            pltpu.sync_copy(rows0, agg_s.at[dstx_v.at[0]], add=True)
            if compute_deg:
                pltpu.sync_copy(ones_v, deg_s.at[dstx_v.at[0]], add=True)

        plsc.subcore_barrier()

        pltpu.sync_copy(agg_s.at[pl.ds(r0, ROWS_T)],
                        agg_out.at[c, pl.ds(r0, ROWS_T)])

        @pl.when(s == 0)
        def _():
            pltpu.sync_copy(agg_s.at[pl.ds(ROWS_T * NS, ROWS_TAIL)],
                            agg_out.at[c, pl.ds(ROWS_T * NS, ROWS_TAIL)])

        if compute_deg:
            @pl.when(s == 0)
            def _():
                pltpu.sync_copy(deg_s, deg_out.at[c])

    return pl.kernel(body, out_type=out_type, scratch_types=scratch, mesh=mesh)


_sc_agg_deg = _make_sc_agg(True)
_sc_agg = _make_sc_agg(False)


RB = 2000
NB = N // RB


def _combine(aggp, degp, xw, wl):
    agg = aggp[0] + aggp[1]
    deg = jnp.maximum(degp[:, 0:1] + degp[:, 1:2], 1.0)
    mean = agg / deg
    return (lax.dot_general(mean, wl[...], (((1,), (1,)), ((), ())),
                            preferred_element_type=jnp.float32)
            + xw[...])


def _tc_hidden_body(aggp, degp, xw, wl, o):
    z = _combine(aggp, degp[...], xw, wl)
    o[...] = jnp.maximum(z, 0.0)


def _tc_final_body(aggp, degp, xw, wl, o):
    z = _combine(aggp, degp[...], xw, wl)
    m = jnp.max(z, axis=-1, keepdims=True)
    lse = jnp.log(jnp.sum(jnp.exp(z - m), axis=-1, keepdims=True)) + m
    o[...] = z - lse


def _tc_layer(body, aggp, degp, xw, wl):
    return pl.pallas_call(
        body,
        grid=(NB,),
        in_specs=[
            pl.BlockSpec((NC, RB, D), lambda i: (0, i, 0)),
            pl.BlockSpec((RB, NC), lambda i: (i, 0)),
            pl.BlockSpec((RB, D), lambda i: (i, 0)),
            pl.BlockSpec((D, D), lambda i: (0, 0)),
        ],
        out_specs=pl.BlockSpec((RB, D), lambda i: (i, 0)),
        out_shape=jax.ShapeDtypeStruct((N, D), jnp.float32),
    )(aggp, degp, xw, wl)


def _tc_xw_body(x, wr, bl, o):
    o[...] = lax.dot_general(x[...], wr[...], (((1,), (1,)), ((), ())),
                             preferred_element_type=jnp.float32) + bl[...]


def _tc_xw(x, wr, bl):
    return pl.pallas_call(
        _tc_xw_body,
        grid=(NB,),
        in_specs=[
            pl.BlockSpec((RB, D), lambda i: (i, 0)),
            pl.BlockSpec((D, D), lambda i: (0, 0)),
            pl.BlockSpec((1, D), lambda i: (0, 0)),
        ],
        out_specs=pl.BlockSpec((RB, D), lambda i: (i, 0)),
        out_shape=jax.ShapeDtypeStruct((N, D), jnp.float32),
    )(x, wr, bl)


def kernel(x, edge_index, Wl1, bl1, Wr1, Wl2, bl2, Wr2):
    ei = edge_index.astype(jnp.int32)

    aggp1, degp = _sc_agg_deg(x, ei)
    xw1 = _tc_xw(x, Wr1, bl1.reshape(1, D))
    degpt = degp.T
    h = _tc_layer(_tc_hidden_body, aggp1, degpt, xw1, Wl1)
    (aggp2,) = _sc_agg(h, ei)
    hw2 = _tc_xw(h, Wr2, bl2.reshape(1, D))
    out = _tc_layer(_tc_final_body, aggp2, degpt, hw2, Wl2)
    return out


# speedup vs baseline: 1.0274x; 1.0137x over previous
import functools

import jax
import jax.numpy as jnp
from jax import lax
from jax.experimental import pallas as pl
from jax.experimental.pallas import tpu as pltpu
from jax.experimental.pallas import tpu_sc as plsc

N = 10000
E = 320000
D = 128

NC = 2
NS = 16
NW = NC * NS
CHUNK = 128
ECHUNKS = E // CHUNK
NCHUNK = 78
NEXTRA = ECHUNKS - NCHUNK * NW
NWIN = 3
WCH = NCHUNK // NWIN
WSZ = WCH * CHUNK
ROWS_T = 624
ROWS_TAIL = N - ROWS_T * NS
ZB = 2000


def _make_sc_agg(compute_deg: bool):
    out_type = [jax.ShapeDtypeStruct((NC, N, D), jnp.float32)]
    if compute_deg:
        out_type.append(jax.ShapeDtypeStruct((NC, N), jnp.float32))

    scratch = [
        pltpu.VMEM((2, WSZ), jnp.int32),
        pltpu.VMEM((2, WSZ), jnp.int32),
        pltpu.VMEM((1, CHUNK), jnp.int32),
        pltpu.VMEM((1, CHUNK), jnp.int32),
        pltpu.VMEM((CHUNK, D), jnp.float32),
        pltpu.VMEM((CHUNK, D), jnp.float32),
        pltpu.VMEM((CHUNK,), jnp.float32),
        pltpu.VMEM((ZB,), jnp.float32),
        pltpu.VMEM_SHARED((N, D), jnp.float32),
        pltpu.VMEM_SHARED((N,), jnp.float32),
        pltpu.SemaphoreType.DMA,
        pltpu.SemaphoreType.DMA,
        pltpu.SemaphoreType.DMA,
        pltpu.SemaphoreType.DMA,
    ]

    mesh = plsc.VectorSubcoreMesh(core_axis_name="c", subcore_axis_name="s")

    def body(x_hbm, e_hbm, *rest):
        if compute_deg:
            agg_out, deg_out = rest[0], rest[1]
            scr = rest[2:]
        else:
            agg_out = rest[0]
            deg_out = None
            scr = rest[1:]
        (src_v, dst_v, srcx_v, dstx_v, rows0, rows1, ones_v, z1_v,
         agg_s, deg_s, sem0, sem1, semw, semz) = scr

        c = lax.axis_index("c")
        s = lax.axis_index("s")
        wid = s * NC + c
        eofs = pl.multiple_of(
            jnp.where(wid < NEXTRA, wid * (NCHUNK + 1),
                      NEXTRA * (NCHUNK + 1)
                      + (wid - NEXTRA) * NCHUNK) * CHUNK,
            CHUNK)

        z16 = jnp.zeros((16,), jnp.float32)

        pltpu.sync_copy(e_hbm.at[0, pl.ds(eofs, WSZ)], src_v.at[0])
        pltpu.sync_copy(e_hbm.at[1, pl.ds(eofs, WSZ)], dst_v.at[0])
        pltpu.async_copy(x_hbm.at[src_v.at[0, pl.ds(0, CHUNK)]], rows0, sem0)

        def zrow(i, carry):
            for j in range(D // 16):
                rows1[i, pl.ds(j * 16, 16)] = z16
            return carry

        lax.fori_loop(0, CHUNK, zrow, 0)

        r0 = s * ROWS_T
        n_full = ROWS_T // CHUNK
        rem = ROWS_T - n_full * CHUNK
        for t in range(n_full):
            pltpu.async_copy(rows1, agg_s.at[pl.ds(r0 + t * CHUNK, CHUNK)],
                             semz)
        if rem:
            pltpu.async_copy(rows1.at[pl.ds(0, rem)],
                             agg_s.at[pl.ds(r0 + n_full * CHUNK, rem)], semz)

        @pl.when(s == 3)
        def _():
            pltpu.sync_copy(rows1.at[pl.ds(0, ROWS_TAIL)],
                            agg_s.at[pl.ds(ROWS_T * NS, ROWS_TAIL)])

        if compute_deg:
            def zz(i, carry):
                z1_v[pl.ds(i * 16, 16)] = z16
                return carry
            lax.fori_loop(0, ZB // 16, zz, 0)

            one16 = jnp.ones((16,), jnp.float32)
            for j in range(CHUNK // 16):
                ones_v[pl.ds(j * 16, 16)] = one16

            @pl.when(s == 2)
            def _():
                nf = N // ZB
                for t in range(nf):
                    pltpu.sync_copy(z1_v, deg_s.at[pl.ds(t * ZB, ZB)])

        for t in range(n_full):
            pltpu.make_async_copy(rows1, agg_s.at[pl.ds(r0 + t * CHUNK,
                                                        CHUNK)], semz).wait()
        if rem:
            pltpu.make_async_copy(rows1.at[pl.ds(0, rem)],
                                  agg_s.at[pl.ds(r0 + n_full * CHUNK, rem)],
                                  semz).wait()

        plsc.subcore_barrier()

        def idx(v, b, k):
            return v.at[b, pl.ds(k * CHUNK, CHUNK)]

        def gather(b, k, buf, sem):
            return pltpu.async_copy(x_hbm.at[idx(src_v, b, k)], buf, sem)

        def scat(b, k, buf):
            pltpu.sync_copy(buf, agg_s.at[idx(dst_v, b, k)], add=True)
            if compute_deg:
                pltpu.sync_copy(ones_v, deg_s.at[idx(dst_v, b, k)], add=True)

        def make_step(b):
            def step(g, carry):
                k0 = 2 * g
                gather(b, k0 + 1, rows1, sem1)
                pltpu.make_async_copy(x_hbm.at[idx(src_v, b, k0)], rows0,
                                      sem0).wait()
                scat(b, k0, rows0)

                @pl.when(k0 + 2 < WCH)
                def _():
                    gather(b, k0 + 2, rows0, sem0)

                pltpu.make_async_copy(x_hbm.at[idx(src_v, b, k0 + 1)], rows1,
                                      sem1).wait()
                scat(b, k0 + 1, rows1)
                return carry
            return step

        for w in range(NWIN):
            b = w % 2
            if w + 1 < NWIN:
                nb = (w + 1) % 2
                wofs = eofs + (w + 1) * WSZ
                pltpu.async_copy(e_hbm.at[0, pl.ds(wofs, WSZ)],
                                 src_v.at[nb], semw)
                pltpu.async_copy(e_hbm.at[1, pl.ds(wofs, WSZ)],
                                 dst_v.at[nb], semw)
            lax.fori_loop(0, WCH // 2, make_step(b), 0)
            if w + 1 < NWIN:
                nb = (w + 1) % 2
                wofs = eofs + (w + 1) * WSZ
                pltpu.make_async_copy(e_hbm.at[0, pl.ds(wofs, WSZ)],
                                      src_v.at[nb], semw).wait()
                pltpu.make_async_copy(e_hbm.at[1, pl.ds(wofs, WSZ)],
                                      dst_v.at[nb], semw).wait()
                gather(nb, 0, rows0, sem0)

        @pl.when(wid < NEXTRA)
        def _():
            xofs = eofs + NCHUNK * CHUNK
            pltpu.sync_copy(e_hbm.at[0, pl.ds(xofs, CHUNK)], srcx_v.at[0])
            pltpu.sync_copy(e_hbm.at[1, pl.ds(xofs, CHUNK)], dstx_v.at[0])
            pltpu.async_copy(x_hbm.at[srcx_v.at[0]], rows0, sem0).wait()
            pltpu.sync_copy(rows0, agg_s.at[dstx_v.at[0]], add=True)
            if compute_deg:
                pltpu.sync_copy(ones_v, deg_s.at[dstx_v.at[0]], add=True)

        plsc.subcore_barrier()

        pltpu.sync_copy(agg_s.at[pl.ds(r0, ROWS_T)],
                        agg_out.at[c, pl.ds(r0, ROWS_T)])

        @pl.when(s == 3)
        def _():
            pltpu.sync_copy(agg_s.at[pl.ds(ROWS_T * NS, ROWS_TAIL)],
                            agg_out.at[c, pl.ds(ROWS_T * NS, ROWS_TAIL)])

        if compute_deg:
            @pl.when(s == 2)
            def _():
                pltpu.sync_copy(deg_s, deg_out.at[c])

    return pl.kernel(body, out_type=out_type, scratch_types=scratch, mesh=mesh)


_sc_agg_deg = _make_sc_agg(True)
_sc_agg = _make_sc_agg(False)


RB = 5000
NB = N // RB


def _combine(aggp, degp, xw, wl):
    agg = aggp[0] + aggp[1]
    deg = jnp.maximum(degp[:, 0:1] + degp[:, 1:2], 1.0)
    mean = agg / deg
    return (lax.dot_general(mean, wl[...], (((1,), (1,)), ((), ())),
                            preferred_element_type=jnp.float32)
            + xw[...])


def _tc_hidden_body(aggp, degp, xw, wl, o):
    z = _combine(aggp, degp[...], xw, wl)
    o[...] = jnp.maximum(z, 0.0)


def _tc_final_body(aggp, degp, xw, wl, o):
    z = _combine(aggp, degp[...], xw, wl)
    m = jnp.max(z, axis=-1, keepdims=True)
    lse = jnp.log(jnp.sum(jnp.exp(z - m), axis=-1, keepdims=True)) + m
    o[...] = z - lse


def _tc_layer(body, aggp, degp, xw, wl):
    return pl.pallas_call(
        body,
        grid=(NB,),
        in_specs=[
            pl.BlockSpec((NC, RB, D), lambda i: (0, i, 0)),
            pl.BlockSpec((RB, NC), lambda i: (i, 0)),
            pl.BlockSpec((RB, D), lambda i: (i, 0)),
            pl.BlockSpec((D, D), lambda i: (0, 0)),
        ],
        out_specs=pl.BlockSpec((RB, D), lambda i: (i, 0)),
        out_shape=jax.ShapeDtypeStruct((N, D), jnp.float32),
    )(aggp, degp, xw, wl)


def _tc_xw_body(x, wr, bl, o):
    o[...] = lax.dot_general(x[...], wr[...], (((1,), (1,)), ((), ())),
                             preferred_element_type=jnp.float32) + bl[...]


def _tc_xw(x, wr, bl):
    return pl.pallas_call(
        _tc_xw_body,
        grid=(NB,),
        in_specs=[
            pl.BlockSpec((RB, D), lambda i: (i, 0)),
            pl.BlockSpec((D, D), lambda i: (0, 0)),
            pl.BlockSpec((1, D), lambda i: (0, 0)),
        ],
        out_specs=pl.BlockSpec((RB, D), lambda i: (i, 0)),
        out_shape=jax.ShapeDtypeStruct((N, D), jnp.float32),
    )(x, wr, bl)


def kernel(x, edge_index, Wl1, bl1, Wr1, Wl2, bl2, Wr2):
    ei = edge_index.astype(jnp.int32)

    aggp1, degp = _sc_agg_deg(x, ei)
    xw1 = _tc_xw(x, Wr1, bl1.reshape(1, D))
    degpt = degp.T
    h = _tc_layer(_tc_hidden_body, aggp1, degpt, xw1, Wl1)
    (aggp2,) = _sc_agg(h, ei)
    hw2 = _tc_xw(h, Wr2, bl2.reshape(1, D))
    out = _tc_layer(_tc_final_body, aggp2, degpt, hw2, Wl2)
    return out
